# baseline (speedup 1.0000x reference)
import functools
import jax
import jax.numpy as jnp
from jax import lax
from jax.experimental import pallas as pl
from jax.experimental.pallas import tpu as pltpu
from jax.experimental.pallas import tpu_sc as plsc

D_MODEL = 64
SCALE = 8.0
LANES = 16

NUM_CORES = 2
NUM_SUBCORES = 16
NUM_WORKERS = NUM_CORES * NUM_SUBCORES

BATCH = 4096 * 200
PER_WORKER = BATCH // NUM_WORKERS
CHUNK = 128
NUM_CHUNKS = PER_WORKER // CHUNK

_mesh = plsc.VectorSubcoreMesh(core_axis_name="c", subcore_axis_name="s")


@functools.partial(
    pl.kernel,
    out_type=jax.ShapeDtypeStruct((BATCH, D_MODEL), jnp.float32),
    mesh=_mesh,
    scratch_types=[
        pltpu.VMEM((NUM_CHUNKS, CHUNK), jnp.int32),
        pltpu.VMEM((CHUNK, D_MODEL), jnp.float32),
        pltpu.SemaphoreType.DMA,
    ],
    compiler_params=pltpu.CompilerParams(use_tc_tiling_on_sc=False),
)
def _embed(idx_hbm, table_hbm, out_hbm, idx_v, rows_v, sem):
    wid = lax.axis_index("s") * NUM_CORES + lax.axis_index("c")
    pltpu.sync_copy(idx_hbm.at[wid], idx_v)

    def chunk_body(g, carry):
        pltpu.async_copy(table_hbm.at[idx_v.at[g]], rows_v, sem).wait()

        def row_body(r, c2):
            for j in range(D_MODEL // LANES):
                sl = pl.ds(j * LANES, LANES)
                rows_v[r, sl] = rows_v[r, sl] * SCALE
            return c2

        lax.fori_loop(0, CHUNK, row_body, 0)
        pltpu.sync_copy(rows_v, out_hbm.at[pl.ds(wid * PER_WORKER + g * CHUNK, CHUNK)])
        return carry

    lax.fori_loop(0, NUM_CHUNKS, chunk_body, 0)


def kernel(x, table):
    xs = x.astype(jnp.int32).reshape(NUM_WORKERS, NUM_CHUNKS, CHUNK)
    out = _embed(xs, table)
    return out.reshape(x.shape + (D_MODEL,))


# speedup vs baseline: 1.2005x; 1.2005x over previous
import functools
import jax
import jax.numpy as jnp
from jax import lax
from jax.experimental import pallas as pl
from jax.experimental.pallas import tpu as pltpu
from jax.experimental.pallas import tpu_sc as plsc

D_MODEL = 64
SCALE = 8.0
LANES = 16

NUM_CORES = 2
NUM_SUBCORES = 16
NUM_WORKERS = NUM_CORES * NUM_SUBCORES

BATCH = 4096 * 200
PER_WORKER = BATCH // NUM_WORKERS
CHUNK = 128
NUM_CHUNKS = PER_WORKER // CHUNK
NBUF = 4
LOOKAHEAD = 2

_mesh = plsc.VectorSubcoreMesh(core_axis_name="c", subcore_axis_name="s")


@functools.partial(
    pl.kernel,
    out_type=jax.ShapeDtypeStruct((BATCH, D_MODEL), jnp.float32),
    mesh=_mesh,
    scratch_types=[
        pltpu.VMEM((NUM_CHUNKS, CHUNK), jnp.int32),
        [pltpu.VMEM((CHUNK, D_MODEL), jnp.float32)] * NBUF,
        [pltpu.SemaphoreType.DMA] * NBUF,
        [pltpu.SemaphoreType.DMA] * NBUF,
    ],
    compiler_params=pltpu.CompilerParams(use_tc_tiling_on_sc=False),
)
def _embed(idx_hbm, table_hbm, out_hbm, idx_v, rows, gsem, ssem):
    wid = lax.axis_index("s") * NUM_CORES + lax.axis_index("c")
    out_base = wid * PER_WORKER
    pltpu.sync_copy(idx_hbm.at[wid], idx_v)

    def gather(g, b):
        return pltpu.async_copy(table_hbm.at[idx_v.at[g]], rows[b], gsem[b])

    def scatter(g, b):
        dst = out_hbm.at[pl.ds(out_base + g * CHUNK, CHUNK)]
        return pltpu.make_async_copy(rows[b], dst, ssem[b])

    for b in range(LOOKAHEAD):
        gather(b, b)

    def outer(i, carry):
        g0 = i * NBUF
        for b in range(NBUF):
            g = g0 + b
            b2 = (b + LOOKAHEAD) % NBUF
            gl = g + LOOKAHEAD

            @pl.when(jnp.logical_and(gl >= NBUF, gl < NUM_CHUNKS))
            def _():
                scatter(gl - NBUF, b2).wait()

            @pl.when(gl < NUM_CHUNKS)
            def _():
                gather(gl, b2)

            pltpu.make_async_copy(table_hbm.at[idx_v.at[g]], rows[b], gsem[b]).wait()

            def row_body(r, c2):
                for j in range(D_MODEL // LANES):
                    sl = pl.ds(j * LANES, LANES)
                    rows[b][r, sl] = rows[b][r, sl] * SCALE
                return c2

            lax.fori_loop(0, CHUNK, row_body, 0)
            scatter(g, b).start()
        return carry

    lax.fori_loop(0, NUM_CHUNKS // NBUF, outer, 0)

    for b in range(NBUF):
        g = NUM_CHUNKS - NBUF + b
        scatter(g, b).wait()


def kernel(x, table):
    xs = x.astype(jnp.int32).reshape(NUM_WORKERS, NUM_CHUNKS, CHUNK)
    out = _embed(xs, table)
    return out.reshape(x.shape + (D_MODEL,))


# speedup vs baseline: 1.2051x; 1.0039x over previous
import functools
import jax
import jax.numpy as jnp
from jax import lax
from jax.experimental import pallas as pl
from jax.experimental.pallas import tpu as pltpu
from jax.experimental.pallas import tpu_sc as plsc

D_MODEL = 64
SCALE = 8.0
LANES = 16

NUM_CORES = 2
NUM_SUBCORES = 16
NUM_WORKERS = NUM_CORES * NUM_SUBCORES

BATCH = 4096 * 200
PER_WORKER = BATCH // NUM_WORKERS
CHUNK = 128
NUM_CHUNKS = PER_WORKER // CHUNK
NBUF = 4
LOOKAHEAD = 2

_mesh = plsc.VectorSubcoreMesh(core_axis_name="c", subcore_axis_name="s")


@functools.partial(
    pl.kernel,
    out_type=jax.ShapeDtypeStruct((BATCH, D_MODEL), jnp.float32),
    mesh=_mesh,
    scratch_types=[
        pltpu.VMEM((NUM_CHUNKS, CHUNK), jnp.int32),
        [pltpu.VMEM((CHUNK, D_MODEL), jnp.float32)] * NBUF,
        [pltpu.SemaphoreType.DMA] * NBUF,
        [pltpu.SemaphoreType.DMA] * NBUF,
    ],
    compiler_params=pltpu.CompilerParams(use_tc_tiling_on_sc=False),
)
def _embed(idx_hbm, table_hbm, out_hbm, idx_v, rows, gsem, ssem):
    wid = lax.axis_index("s") * NUM_CORES + lax.axis_index("c")
    out_base = wid * PER_WORKER
    pltpu.sync_copy(idx_hbm.at[wid], idx_v)

    def gather(g, b):
        return pltpu.async_copy(table_hbm.at[idx_v.at[g]], rows[b], gsem[b])

    def scatter(g, b):
        dst = out_hbm.at[pl.ds(out_base + g * CHUNK, CHUNK)]
        return pltpu.make_async_copy(rows[b], dst, ssem[b])

    for b in range(LOOKAHEAD):
        gather(b, b)

    def outer(i, carry):
        g0 = i * NBUF
        for b in range(NBUF):
            g = g0 + b
            b2 = (b + LOOKAHEAD) % NBUF
            gl = g + LOOKAHEAD

            @pl.when(jnp.logical_and(gl >= NBUF, gl < NUM_CHUNKS))
            def _():
                scatter(gl - NBUF, b2).wait()

            @pl.when(gl < NUM_CHUNKS)
            def _():
                gather(gl, b2)

            pltpu.make_async_copy(table_hbm.at[idx_v.at[g]], rows[b], gsem[b]).wait()

            @plsc.parallel_loop(0, CHUNK, step=1, unroll=8)
            def _(r):
                for j in range(D_MODEL // LANES):
                    sl = pl.ds(j * LANES, LANES)
                    rows[b][r, sl] = rows[b][r, sl] * SCALE
            scatter(g, b).start()
        return carry

    lax.fori_loop(0, NUM_CHUNKS // NBUF, outer, 0)

    for b in range(NBUF):
        g = NUM_CHUNKS - NBUF + b
        scatter(g, b).wait()


def kernel(x, table):
    xs = x.astype(jnp.int32).reshape(NUM_WORKERS, NUM_CHUNKS, CHUNK)
    out = _embed(xs, table)
    return out.reshape(x.shape + (D_MODEL,))
